# baseline (speedup 1.0000x reference)
import numpy as np
import jax
import jax.numpy as jnp
from jax import lax
from jax.experimental import pallas as pl
from jax.experimental.pallas import tpu as pltpu
from jax.experimental.pallas import tpu_sc as plsc

_NUM_FIELDS = 26
_VOCAB = 100000
_EMBED = 16
_BATCH = 16384
_NC = 2
_NS = 16
_FPC = _NUM_FIELDS // _NC
_NB = 4096
_NCHUNK = _BATCH // _NB
_FW = 100352
_NQ = 4
_FQ = _FW // _NQ
_LAST_C0A = (25 * _VOCAB // 128) * 128
_LAST_W = 2600000 - _LAST_C0A

_DELTAS = np.array([(f % 4) * 32 for f in range(_NUM_FIELDS)], np.int32)


def _sc_body(xlin_hbm, tt_hbm, tail_hbm, out_hbm, sub_v, xb_v, ob_v,
             rsem, xsem, osem):
    cid = lax.axis_index("c")
    sid = lax.axis_index("s")

    def xb_start(f, cc, buf):
        return pltpu.async_copy(
            xlin_hbm.at[pl.ds(f * _BATCH + cc * _NB, _NB)],
            xb_v.at[buf], xsem.at[buf])

    def row_start(src_hbm, base):
        return [pltpu.async_copy(
            src_hbm.at[sid, pl.ds(pl.multiple_of(base + q * _FQ, 128), _FQ)],
            sub_v.at[pl.ds(q * _FQ, _FQ)], rsem.at[q])
            for q in range(_NQ)]

    xh = [None, None]
    oh = [None, None]
    xh[0] = xb_start(cid * _FPC, 0, 0)

    for k in range(_FPC):
        f = cid * _FPC + k
        c0a = pl.multiple_of(f * _VOCAB - (f % 4) * 32, 128)

        if k < _FPC - 1:
            rh = row_start(tt_hbm, c0a)
        else:
            @pl.when(cid == 0)
            def _():
                row_start(tt_hbm, c0a)

            @pl.when(cid == 1)
            def _():
                row_start(tail_hbm, 0)

            rh = None

        for cc in range(_NCHUNK):
            buf = cc & 1
            nbuf = (cc + 1) & 1
            if cc + 1 < _NCHUNK:
                xh[nbuf] = xb_start(f, cc + 1, nbuf)
            elif k + 1 < _FPC:
                xh[nbuf] = xb_start(f + 1, 0, nbuf)

            if cc == 0:
                if rh is not None:
                    for h in rh:
                        h.wait()
                else:
                    for q in range(_NQ):
                        pltpu.make_async_copy(
                            tt_hbm.at[sid, pl.ds(pl.multiple_of(
                                c0a + q * _FQ, 128), _FQ)],
                            sub_v.at[pl.ds(q * _FQ, _FQ)],
                            rsem.at[q]).wait()
            xh[buf].wait()
            if oh[buf] is not None:
                oh[buf].wait()

            def chunk_body(j, carry):
                base = j * 256
                for u in range(16):
                    s = pl.ds(base + u * 16, 16)
                    ob_v[buf, s] = plsc.load_gather(sub_v, [xb_v[buf, s]])
                return carry

            lax.fori_loop(0, _NB // 256, chunk_body, 0)
            oh[buf] = pltpu.async_copy(
                ob_v.at[buf],
                out_hbm.at[pl.ds(f * (_EMBED * _BATCH) + sid * _BATCH
                                 + cc * _NB, _NB)],
                osem.at[buf])

    for h in oh:
        if h is not None:
            h.wait()


@jax.jit
def kernel(x, table):
    mesh = plsc.VectorSubcoreMesh(core_axis_name="c", subcore_axis_name="s")
    run = pl.kernel(
        _sc_body,
        mesh=mesh,
        out_type=jax.ShapeDtypeStruct((_NUM_FIELDS * _EMBED * _BATCH,),
                                      jnp.float32),
        scratch_types=[
            pltpu.VMEM((_FW,), jnp.float32),
            pltpu.VMEM((2, _NB), jnp.int32),
            pltpu.VMEM((2, _NB), jnp.float32),
            pltpu.SemaphoreType.DMA((_NQ,)),
            pltpu.SemaphoreType.DMA((2,)),
            pltpu.SemaphoreType.DMA((2,)),
        ],
        compiler_params=pltpu.CompilerParams(needs_layout_passes=False),
    )
    tt = table.T
    tail = jnp.pad(tt[:, _LAST_C0A:], ((0, 0), (0, _FW - _LAST_W)))
    xlin = (x + _DELTAS[None, :]).T.reshape(_NUM_FIELDS * _BATCH)
    out = run(xlin, tt, tail)
    return jnp.transpose(out.reshape(_NUM_FIELDS, _EMBED, _BATCH), (2, 0, 1))


# speedup vs baseline: 1.0112x; 1.0112x over previous
import numpy as np
import jax
import jax.numpy as jnp
from jax import lax
from jax.experimental import pallas as pl
from jax.experimental.pallas import tpu as pltpu
from jax.experimental.pallas import tpu_sc as plsc

_NUM_FIELDS = 26
_VOCAB = 100000
_EMBED = 16
_BATCH = 16384
_NC = 2
_NS = 16
_FPC = _NUM_FIELDS // _NC
_NB = 4096
_NCHUNK = _BATCH // _NB
_FW = 100096
_FH = _FW // 2
_SUBW = 100224
_LAST_C0A = (25 * _VOCAB // 128) * 128
_LAST_W = 2600000 - _LAST_C0A

_DELTAS = np.array([(f % 4) * 32 for f in range(_NUM_FIELDS)], np.int32)


def _sc_body(xlin_hbm, tt_hbm, tail_hbm, out_hbm, sub_v, xb_v, ob_v,
             rsem, xsem, osem):
    cid = lax.axis_index("c")
    sid = lax.axis_index("s")

    def xb_start(f, cc, buf):
        return pltpu.async_copy(
            xlin_hbm.at[pl.ds(f * _BATCH + cc * _NB, _NB)],
            xb_v.at[buf], xsem.at[buf])

    xh = [None, None]
    oh = [None, None]
    xh[0] = xb_start(cid * _FPC, 0, 0)

    for k in range(_FPC):
        f = cid * _FPC + k
        c0a = pl.multiple_of(f * _VOCAB - (f % 4) * 32, 128)

        rh = []
        if k < _FPC - 1:
            rh.append(pltpu.async_copy(
                tt_hbm.at[sid, pl.ds(c0a, _FH)],
                sub_v.at[pl.ds(0, _FH)], rsem.at[0]))
            rh.append(pltpu.async_copy(
                tt_hbm.at[sid, pl.ds(c0a + _FH, _FH)],
                sub_v.at[pl.ds(_FH, _FH)], rsem.at[1]))
        else:
            @pl.when(cid == 0)
            def _():
                pltpu.sync_copy(tt_hbm.at[sid, pl.ds(c0a, _FW)],
                                sub_v.at[pl.ds(0, _FW)])

            @pl.when(cid == 1)
            def _():
                pltpu.sync_copy(tail_hbm.at[sid, pl.ds(0, _SUBW)], sub_v)

        for cc in range(_NCHUNK):
            buf = cc & 1
            nbuf = (cc + 1) & 1
            if cc + 1 < _NCHUNK:
                xh[nbuf] = xb_start(f, cc + 1, nbuf)
            elif k + 1 < _FPC:
                xh[nbuf] = xb_start(f + 1, 0, nbuf)

            if cc == 0:
                for h in rh:
                    h.wait()
            xh[buf].wait()
            if oh[buf] is not None:
                oh[buf].wait()

            def chunk_body(j, carry):
                base = j * 256
                for u in range(16):
                    s = pl.ds(base + u * 16, 16)
                    ob_v[buf, s] = plsc.load_gather(sub_v, [xb_v[buf, s]])
                return carry

            lax.fori_loop(0, _NB // 256, chunk_body, 0)
            oh[buf] = pltpu.async_copy(
                ob_v.at[buf],
                out_hbm.at[pl.ds(f * (_EMBED * _BATCH) + sid * _BATCH
                                 + cc * _NB, _NB)],
                osem.at[buf])

    for h in oh:
        if h is not None:
            h.wait()


@jax.jit
def kernel(x, table):
    mesh = plsc.VectorSubcoreMesh(core_axis_name="c", subcore_axis_name="s")
    run = pl.kernel(
        _sc_body,
        mesh=mesh,
        out_type=jax.ShapeDtypeStruct((_NUM_FIELDS * _EMBED * _BATCH,),
                                      jnp.float32),
        scratch_types=[
            pltpu.VMEM((_SUBW,), jnp.float32),
            pltpu.VMEM((2, _NB), jnp.int32),
            pltpu.VMEM((2, _NB), jnp.float32),
            pltpu.SemaphoreType.DMA((2,)),
            pltpu.SemaphoreType.DMA((2,)),
            pltpu.SemaphoreType.DMA((2,)),
        ],
        compiler_params=pltpu.CompilerParams(needs_layout_passes=False),
    )
    tt = table.T
    tail = jnp.pad(tt[:, _LAST_C0A:], ((0, 0), (0, _SUBW - _LAST_W)))
    xlin = (x + _DELTAS[None, :]).T.reshape(_NUM_FIELDS * _BATCH)
    out = run(xlin, tt, tail)
    return jnp.transpose(out.reshape(_NUM_FIELDS, _EMBED, _BATCH), (2, 0, 1))
